# Initial kernel scaffold; baseline (speedup 1.0000x reference)
import jax
import jax.numpy as jnp
from jax.experimental import pallas as pl

N_CLASS = 500
N_DIM = 10
BATCH = 16384


def _fused_kernel(x_ref, k_ref, w_ref, b_ref, out_ref, loss_ref):
    k = k_ref[...]
    w = w_ref[...]
    w_eff = jnp.sum(k * w[:, 0][None, :], axis=1)
    x = x_ref[...]
    out_ref[...] = jnp.sum(x * w_eff[None, :], axis=1, keepdims=True) + b_ref[0, 0]
    mean = jnp.mean(k, axis=1, keepdims=True)
    cent = k - mean
    loss_ref[0, 0] = jnp.sum(cent * cent) / (N_CLASS - 1.0)


def kernel(inputs, dense_cov_kernel, final_w, final_b, segment_ids):
    del segment_ids
    b = final_b.reshape(1, 1)
    out, loss = pl.pallas_call(
        _fused_kernel,
        out_shape=(
            jax.ShapeDtypeStruct((BATCH, 1), jnp.float32),
            jax.ShapeDtypeStruct((1, 1), jnp.float32),
        ),
    )(inputs, dense_cov_kernel, final_w, b)
    return out, loss[0, 0]


# speedup vs baseline: 2.3289x; 2.3289x over previous
import jax
import jax.numpy as jnp
from jax.experimental import pallas as pl

N_CLASS = 500
N_DIM = 10
BATCH = 16384


def _fused_kernel(x_ref, k_ref, w_ref, b_ref, out_ref, loss_ref):
    k = k_ref[...]
    w = w_ref[...]
    w_eff = jnp.sum(k * w[:, 0][None, :], axis=1)
    x = x_ref[...]
    out_ref[...] = jnp.sum(x * w_eff[None, :], axis=1, keepdims=True) + b_ref[...]
    mean = jnp.mean(k, axis=1, keepdims=True)
    cent = k - mean
    loss_ref[...] = (jnp.sum(cent * cent) / (N_CLASS - 1.0)).reshape(1, 1)


def kernel(inputs, dense_cov_kernel, final_w, final_b, segment_ids):
    del segment_ids
    b = final_b.reshape(1, 1)
    out, loss = pl.pallas_call(
        _fused_kernel,
        out_shape=(
            jax.ShapeDtypeStruct((BATCH, 1), jnp.float32),
            jax.ShapeDtypeStruct((1, 1), jnp.float32),
        ),
    )(inputs, dense_cov_kernel, final_w, b)
    return out, loss[0, 0]
